# baseline (speedup 1.0000x reference)
import functools

import jax
import jax.numpy as jnp
from jax import lax
from jax.experimental import pallas as pl
from jax.experimental.pallas import tpu as pltpu
from jax.experimental.pallas import tpu_sc as plsc

N_NODES = 50000
N_EDGES = 800000
P = 8
NC = 2
NS = 16
NW = NC * NS
L = 16

U = 256
G = U // L
GB = 128
NUNITS = N_EDGES // U
MAIN = NUNITS // NW
EXTRA_W = NUNITS - MAIN * NW

WB_BYTES = U * P * 4 * 4 + U * P * 4


def _splat(v):
    return jnp.full((L,), v, dtype=jnp.int32)


def _i32(v):
    return jnp.int32(v)


def _fori(n, body):
    lax.fori_loop(_i32(0), _i32(n), body, _i32(0))


def _sc_body(ptab, ec, wts, ratios, rmw,
             ecv0, ecv1, eiv0, eiv1, ejv0, ejv1,
             xiv0, xiv1, xjv0, xjv1, outv0, outv1, wv, w8v,
             sem_idx0, sem_idx1, sem_g0, sem_g1, sem_wb0, sem_wb1):
    wid = lax.axis_index("s") * NC + lax.axis_index("c")
    start_u = wid * _i32(MAIN) + jnp.minimum(wid, _i32(EXTRA_W))
    iota16 = lax.iota(jnp.int32, L)
    zero16 = jnp.zeros((L,), dtype=jnp.int32)
    c0s, c1s = _splat(0), _splat(1)

    def ebase(c):
        return (start_u + c) * _i32(U)

    def idx_issue(c, ecv, sem):
        pltpu.async_copy(ec.at[pl.ds(ebase(c), U)], ecv, sem)

    def idx_wait(ecv, sem):
        pltpu.make_async_copy(ec.at[pl.ds(0, U)], ecv, sem).wait()

    def dec(ecv, eiv, ejv):
        @plsc.parallel_loop(_i32(0), _i32(G), step=_i32(1), unroll=4)
        def _(g):
            e16 = g * _i32(L) + iota16
            eiv[pl.ds(g * _i32(L), L)] = plsc.load_gather(ecv, [e16, c0s])
            ejv[pl.ds(g * _i32(L), L)] = plsc.load_gather(ecv, [e16, c1s])

    def gath_issue(eiv, ejv, xiv, xjv, sem):
        for s in (0, GB):
            pltpu.async_copy(ptab.at[eiv.at[pl.ds(s, GB)]],
                             xiv.at[pl.ds(s, GB)], sem)
            pltpu.async_copy(ptab.at[ejv.at[pl.ds(s, GB)]],
                             xjv.at[pl.ds(s, GB)], sem)

    def gath_wait(eiv, ejv, xiv, xjv, sem):
        for s in (0, GB):
            pltpu.make_async_copy(ptab.at[eiv.at[pl.ds(s, GB)]],
                                  xiv.at[pl.ds(s, GB)], sem).wait()
            pltpu.make_async_copy(ptab.at[ejv.at[pl.ds(s, GB)]],
                                  xjv.at[pl.ds(s, GB)], sem).wait()

    def comp(xiv, xjv, outv):
        @plsc.parallel_loop(_i32(0), _i32(G), step=_i32(1), unroll=2)
        def _(g):
            e16 = g * _i32(L) + iota16
            for p in range(P):
                b = 4 * p
                ps = _splat(p)
                w1 = plsc.load_gather(xiv, [e16, _splat(b)])
                x1 = plsc.load_gather(xiv, [e16, _splat(b + 1)])
                y1 = plsc.load_gather(xiv, [e16, _splat(b + 2)])
                z1 = plsc.load_gather(xiv, [e16, _splat(b + 3)])
                w2 = plsc.load_gather(xjv, [e16, _splat(b)])
                x2 = plsc.load_gather(xjv, [e16, _splat(b + 1)])
                y2 = plsc.load_gather(xjv, [e16, _splat(b + 2)])
                z2 = plsc.load_gather(xjv, [e16, _splat(b + 3)])
                rw = (w1 * w2 + x1 * x2) + (y1 * y2 + z1 * z2)
                rx = (x1 * w2 - w1 * x2) + (z1 * y2 - y1 * z2)
                ry = (y1 * w2 - w1 * y2) + (x1 * z2 - z1 * x2)
                rz = (z1 * w2 - w1 * z2) + (y1 * x2 - x1 * y2)
                plsc.store_scatter(outv, [e16, ps, c0s], rw)
                plsc.store_scatter(outv, [e16, ps, c1s], rx)
                plsc.store_scatter(outv, [e16, ps, _splat(2)], ry)
                plsc.store_scatter(outv, [e16, ps, _splat(3)], rz)

    def wb_issue(outv, c, sem):
        b = ebase(c)
        pltpu.async_copy(outv, ratios.at[pl.ds(b, U)], sem)
        pltpu.async_copy(wv, rmw.at[pl.ds(b, U)], sem)

    def wb_wait(outv, sem):
        pltpu.make_async_copy(ratios.at[pl.ds(0, U)], outv, sem).wait()
        pltpu.make_async_copy(rmw.at[pl.ds(0, U)], wv, sem).wait()

    pltpu.sync_copy(wts, w8v)
    wvals = plsc.load_gather(w8v, [zero16, iota16 & _i32(7)])

    def wfill(s, carry):
        r0 = s * _i32(2) + (iota16 >> _i32(3))
        plsc.store_scatter(wv, [r0, iota16 & _i32(7)], wvals)
        return carry
    _fori(U * P // L, wfill)

    pltpu.sync_copy(ec.at[pl.ds(ebase(_i32(0)), U)], ecv0)
    dec(ecv0, eiv0, ejv0)
    gath_issue(eiv0, ejv0, xiv0, xjv0, sem_g0)
    idx_issue(_i32(1), ecv1, sem_idx1)

    def pair(d, carry):
        c = d * _i32(2)
        idx_issue(c + _i32(2), ecv0, sem_idx0)
        idx_wait(ecv1, sem_idx1)
        dec(ecv1, eiv1, ejv1)
        gath_issue(eiv1, ejv1, xiv1, xjv1, sem_g1)

        @pl.when(d >= _i32(1))
        def _():
            wb_wait(outv0, sem_wb0)
        gath_wait(eiv0, ejv0, xiv0, xjv0, sem_g0)
        comp(xiv0, xjv0, outv0)
        wb_issue(outv0, c, sem_wb0)

        @pl.when(d < _i32(MAIN // 2 - 1))
        def _():
            idx_issue(c + _i32(3), ecv1, sem_idx1)
        idx_wait(ecv0, sem_idx0)
        dec(ecv0, eiv0, ejv0)
        gath_issue(eiv0, ejv0, xiv0, xjv0, sem_g0)

        @pl.when(d >= _i32(1))
        def _():
            wb_wait(outv1, sem_wb1)
        gath_wait(eiv1, ejv1, xiv1, xjv1, sem_g1)
        comp(xiv1, xjv1, outv1)
        wb_issue(outv1, c + _i32(1), sem_wb1)
        return carry
    _fori(MAIN // 2, pair)

    last = _i32(MAIN - 1)
    wb_wait(outv0, sem_wb0)
    gath_wait(eiv0, ejv0, xiv0, xjv0, sem_g0)
    comp(xiv0, xjv0, outv0)
    wb_issue(outv0, last, sem_wb0)

    t = jnp.where(wid < _i32(EXTRA_W), _i32(MAIN), last)
    pltpu.sync_copy(ec.at[pl.ds(ebase(t), U)], ecv1)
    dec(ecv1, eiv1, ejv1)
    gath_issue(eiv1, ejv1, xiv1, xjv1, sem_g1)
    wb_wait(outv1, sem_wb1)
    gath_wait(eiv1, ejv1, xiv1, xjv1, sem_g1)
    comp(xiv1, xjv1, outv1)
    wb_issue(outv1, t, sem_wb1)

    wb_wait(outv0, sem_wb0)
    wb_wait(outv1, sem_wb1)


@functools.partial(
    pl.kernel,
    out_type=(jax.ShapeDtypeStruct((N_EDGES, P, 4), jnp.float32),
              jax.ShapeDtypeStruct((N_EDGES, P), jnp.float32)),
    mesh=plsc.VectorSubcoreMesh(core_axis_name="c", subcore_axis_name="s",
                                num_cores=NC, num_subcores=NS),
    compiler_params=pltpu.CompilerParams(needs_layout_passes=False,
                                         use_tc_tiling_on_sc=False),
    scratch_types=[
        pltpu.VMEM((U, 2), jnp.int32),
        pltpu.VMEM((U, 2), jnp.int32),
        pltpu.VMEM((U,), jnp.int32),
        pltpu.VMEM((U,), jnp.int32),
        pltpu.VMEM((U,), jnp.int32),
        pltpu.VMEM((U,), jnp.int32),
        pltpu.VMEM((U, 4 * P), jnp.float32),
        pltpu.VMEM((U, 4 * P), jnp.float32),
        pltpu.VMEM((U, 4 * P), jnp.float32),
        pltpu.VMEM((U, 4 * P), jnp.float32),
        pltpu.VMEM((U, P, 4), jnp.float32),
        pltpu.VMEM((U, P, 4), jnp.float32),
        pltpu.VMEM((U, P), jnp.float32),
        pltpu.VMEM((1, P), jnp.float32),
        pltpu.SemaphoreType.DMA,
        pltpu.SemaphoreType.DMA,
        pltpu.SemaphoreType.DMA,
        pltpu.SemaphoreType.DMA,
        pltpu.SemaphoreType.DMA,
        pltpu.SemaphoreType.DMA,
    ],
)
def _quat_edges_sc(ptab, ec, wts, ratios, rmw, *scratch):
    _sc_body(ptab, ec, wts, ratios, rmw, *scratch)


def kernel(particles, weights, edges):
    ec = edges.astype(jnp.int32)
    ptab = particles.astype(jnp.float32).reshape(N_NODES, 4 * P)
    return _quat_edges_sc(ptab, ec, weights.astype(jnp.float32))


# speedup vs baseline: 1.4538x; 1.4538x over previous
import functools

import jax
import jax.numpy as jnp
from jax import lax
from jax.experimental import pallas as pl
from jax.experimental.pallas import tpu as pltpu
from jax.experimental.pallas import tpu_sc as plsc

N_NODES = 50000
N_EDGES = 800000
P = 8
NC = 2
NS = 16
NW = NC * NS
L = 16

U = 256
G = U // L
GB = 128
NUNITS = N_EDGES // U
MAIN = NUNITS // NW
EXTRA_W = NUNITS - MAIN * NW

WB_BYTES = U * P * 4 * 4 + U * P * 4


def _splat(v):
    return jnp.full((L,), v, dtype=jnp.int32)


def _i32(v):
    return jnp.int32(v)


def _fori(n, body):
    lax.fori_loop(_i32(0), _i32(n), body, _i32(0))


def _sc_body(ptab, ec, wts, ratios, rmw,
             ecv0, ecv1, eiv0, eiv1, ejv0, ejv1,
             xiv0, xiv1, xjv0, xjv1, outv0, outv1, wv, w8v,
             sem_idx0, sem_idx1, sem_g0, sem_g1, sem_wb0, sem_wb1):
    wid = lax.axis_index("s") * NC + lax.axis_index("c")
    start_u = wid * _i32(MAIN) + jnp.minimum(wid, _i32(EXTRA_W))
    iota16 = lax.iota(jnp.int32, L)
    zero16 = jnp.zeros((L,), dtype=jnp.int32)
    c0s, c1s = _splat(0), _splat(1)

    def ebase(c):
        return (start_u + c) * _i32(U)

    def idx_issue(c, ecv, sem):
        pltpu.async_copy(ec.at[pl.ds(ebase(c), U)], ecv, sem)

    def idx_wait(ecv, sem):
        pltpu.make_async_copy(ec.at[pl.ds(0, U)], ecv, sem).wait()

    def dec(ecv, eiv, ejv):
        @plsc.parallel_loop(_i32(0), _i32(G), step=_i32(1), unroll=4)
        def _(g):
            e16 = g * _i32(L) + iota16
            eiv[pl.ds(g * _i32(L), L)] = plsc.load_gather(ecv, [e16, c0s])
            ejv[pl.ds(g * _i32(L), L)] = plsc.load_gather(ecv, [e16, c1s])

    def gath_issue(eiv, ejv, xiv, xjv, sem):
        for s in (0, GB):
            pltpu.async_copy(ptab.at[eiv.at[pl.ds(s, GB)]],
                             xiv.at[pl.ds(s, GB)], sem)
            pltpu.async_copy(ptab.at[ejv.at[pl.ds(s, GB)]],
                             xjv.at[pl.ds(s, GB)], sem)

    def gath_wait(eiv, ejv, xiv, xjv, sem):
        for s in (0, GB):
            pltpu.make_async_copy(ptab.at[eiv.at[pl.ds(s, GB)]],
                                  xiv.at[pl.ds(s, GB)], sem).wait()
            pltpu.make_async_copy(ptab.at[ejv.at[pl.ds(s, GB)]],
                                  xjv.at[pl.ds(s, GB)], sem).wait()

    def comp(xiv, xjv, outv):
        @plsc.parallel_loop(_i32(0), _i32(G), step=_i32(1), unroll=2)
        def _(g):
            e16 = g * _i32(L) + iota16
            for p in range(P):
                b = 4 * p
                w1 = plsc.load_gather(xiv, [e16, _splat(b)])
                x1 = plsc.load_gather(xiv, [e16, _splat(b + 1)])
                y1 = plsc.load_gather(xiv, [e16, _splat(b + 2)])
                z1 = plsc.load_gather(xiv, [e16, _splat(b + 3)])
                w2 = plsc.load_gather(xjv, [e16, _splat(b)])
                x2 = plsc.load_gather(xjv, [e16, _splat(b + 1)])
                y2 = plsc.load_gather(xjv, [e16, _splat(b + 2)])
                z2 = plsc.load_gather(xjv, [e16, _splat(b + 3)])
                rw = (w1 * w2 + x1 * x2) + (y1 * y2 + z1 * z2)
                rx = (x1 * w2 - w1 * x2) + (z1 * y2 - y1 * z2)
                ry = (y1 * w2 - w1 * y2) + (x1 * z2 - z1 * x2)
                rz = (z1 * w2 - w1 * z2) + (y1 * x2 - x1 * y2)
                plsc.store_scatter(outv, [e16, _splat(b)], rw)
                plsc.store_scatter(outv, [e16, _splat(b + 1)], rx)
                plsc.store_scatter(outv, [e16, _splat(b + 2)], ry)
                plsc.store_scatter(outv, [e16, _splat(b + 3)], rz)

    def wb_issue(outv, c, sem):
        b = ebase(c)
        pltpu.async_copy(outv, ratios.at[pl.ds(b, U)], sem)
        pltpu.async_copy(wv, rmw.at[pl.ds(b * _i32(P), U * P)], sem)

    def wb_wait(outv, sem):
        pltpu.make_async_copy(ratios.at[pl.ds(0, U)], outv, sem).wait()
        pltpu.make_async_copy(rmw.at[pl.ds(0, U * P)], wv, sem).wait()

    pltpu.sync_copy(wts, w8v)
    wvals = plsc.load_gather(w8v, [zero16, iota16 & _i32(7)])

    def wfill(s, carry):
        wv[pl.ds(s * _i32(L), L)] = wvals
        return carry
    _fori(U * P // L, wfill)

    pltpu.sync_copy(ec.at[pl.ds(ebase(_i32(0)), U)], ecv0)
    dec(ecv0, eiv0, ejv0)
    gath_issue(eiv0, ejv0, xiv0, xjv0, sem_g0)
    idx_issue(_i32(1), ecv1, sem_idx1)

    def pair(d, carry):
        c = d * _i32(2)
        idx_issue(c + _i32(2), ecv0, sem_idx0)
        idx_wait(ecv1, sem_idx1)
        dec(ecv1, eiv1, ejv1)
        gath_issue(eiv1, ejv1, xiv1, xjv1, sem_g1)

        @pl.when(d >= _i32(1))
        def _():
            wb_wait(outv0, sem_wb0)
        gath_wait(eiv0, ejv0, xiv0, xjv0, sem_g0)
        comp(xiv0, xjv0, outv0)
        wb_issue(outv0, c, sem_wb0)

        @pl.when(d < _i32(MAIN // 2 - 1))
        def _():
            idx_issue(c + _i32(3), ecv1, sem_idx1)
        idx_wait(ecv0, sem_idx0)
        dec(ecv0, eiv0, ejv0)
        gath_issue(eiv0, ejv0, xiv0, xjv0, sem_g0)

        @pl.when(d >= _i32(1))
        def _():
            wb_wait(outv1, sem_wb1)
        gath_wait(eiv1, ejv1, xiv1, xjv1, sem_g1)
        comp(xiv1, xjv1, outv1)
        wb_issue(outv1, c + _i32(1), sem_wb1)
        return carry
    _fori(MAIN // 2, pair)

    last = _i32(MAIN - 1)
    wb_wait(outv0, sem_wb0)
    gath_wait(eiv0, ejv0, xiv0, xjv0, sem_g0)
    comp(xiv0, xjv0, outv0)
    wb_issue(outv0, last, sem_wb0)

    t = jnp.where(wid < _i32(EXTRA_W), _i32(MAIN), last)
    pltpu.sync_copy(ec.at[pl.ds(ebase(t), U)], ecv1)
    dec(ecv1, eiv1, ejv1)
    gath_issue(eiv1, ejv1, xiv1, xjv1, sem_g1)
    wb_wait(outv1, sem_wb1)
    gath_wait(eiv1, ejv1, xiv1, xjv1, sem_g1)
    comp(xiv1, xjv1, outv1)
    wb_issue(outv1, t, sem_wb1)

    wb_wait(outv0, sem_wb0)
    wb_wait(outv1, sem_wb1)


@functools.partial(
    pl.kernel,
    out_type=(jax.ShapeDtypeStruct((N_EDGES, 4 * P), jnp.float32),
              jax.ShapeDtypeStruct((N_EDGES * P,), jnp.float32)),
    mesh=plsc.VectorSubcoreMesh(core_axis_name="c", subcore_axis_name="s",
                                num_cores=NC, num_subcores=NS),
    compiler_params=pltpu.CompilerParams(needs_layout_passes=False,
                                         use_tc_tiling_on_sc=False),
    scratch_types=[
        pltpu.VMEM((U, 2), jnp.int32),
        pltpu.VMEM((U, 2), jnp.int32),
        pltpu.VMEM((U,), jnp.int32),
        pltpu.VMEM((U,), jnp.int32),
        pltpu.VMEM((U,), jnp.int32),
        pltpu.VMEM((U,), jnp.int32),
        pltpu.VMEM((U, 4 * P), jnp.float32),
        pltpu.VMEM((U, 4 * P), jnp.float32),
        pltpu.VMEM((U, 4 * P), jnp.float32),
        pltpu.VMEM((U, 4 * P), jnp.float32),
        pltpu.VMEM((U, 4 * P), jnp.float32),
        pltpu.VMEM((U, 4 * P), jnp.float32),
        pltpu.VMEM((U * P,), jnp.float32),
        pltpu.VMEM((1, P), jnp.float32),
        pltpu.SemaphoreType.DMA,
        pltpu.SemaphoreType.DMA,
        pltpu.SemaphoreType.DMA,
        pltpu.SemaphoreType.DMA,
        pltpu.SemaphoreType.DMA,
        pltpu.SemaphoreType.DMA,
    ],
)
def _quat_edges_sc(ptab, ec, wts, ratios, rmw, *scratch):
    _sc_body(ptab, ec, wts, ratios, rmw, *scratch)


def kernel(particles, weights, edges):
    ec = edges.astype(jnp.int32)
    ptab = particles.astype(jnp.float32).reshape(N_NODES, 4 * P)
    ratios, rmw = _quat_edges_sc(ptab, ec, weights.astype(jnp.float32))
    return ratios.reshape(N_EDGES, P, 4), rmw.reshape(N_EDGES, P)
